# Initial kernel scaffold; baseline (speedup 1.0000x reference)
import functools

import jax
import jax.numpy as jnp
import numpy as np
from jax import lax
from jax.experimental import pallas as pl
from jax.experimental.pallas import tpu as pltpu
from jax.experimental.pallas import tpu_sc as plsc

N = 2100
NP = 2176
E0 = 33600
E1 = E0 + N
CH = 512
EP = ((E1 + CH - 1) // CH) * CH
SENT = 2160
BM = 128
NBLK = NP // BM
H1DIM = 1024
NTILES = 32
R = 34

_f32 = jnp.float32
_i32 = jnp.int32


def _tc1_body(x_ref, w_ref, ats_ref, atd_ref, h_ref, as_ref, ad_ref, cs_ref):
    i = pl.program_id(0)
    h = jnp.dot(x_ref[...], w_ref[...], preferred_element_type=_f32)
    h_ref[...] = h
    cols = []
    for hd in range(2):
        hh = h[:, hd * H1DIM:(hd + 1) * H1DIM]
        a_s = jnp.sum(hh * ats_ref[hd:hd + 1, :], axis=1, keepdims=True)
        a_d = jnp.sum(hh * atd_ref[hd:hd + 1, :], axis=1, keepdims=True)
        cols.append((a_s, a_d))
    a_s = jnp.concatenate([cols[0][0], cols[1][0]], axis=1)
    a_d = jnp.concatenate([cols[0][1], cols[1][1]], axis=1)
    as_ref[...] = a_s
    ad_ref[...] = a_d
    m = jnp.concatenate([jnp.max(a_s, axis=0, keepdims=True),
                         jnp.max(a_d, axis=0, keepdims=True)], axis=1)

    @pl.when(i == 0)
    def _():
        cs_ref[...] = m

    @pl.when(i > 0)
    def _():
        cs_ref[...] = jnp.maximum(cs_ref[...], m)


def _tc1(x_p, w1, ats, atd):
    return pl.pallas_call(
        _tc1_body,
        grid=(NBLK,),
        in_specs=[
            pl.BlockSpec((BM, 2048), lambda i: (i, 0)),
            pl.BlockSpec((2048, 2048), lambda i: (0, 0)),
            pl.BlockSpec((2, H1DIM), lambda i: (0, 0)),
            pl.BlockSpec((2, H1DIM), lambda i: (0, 0)),
        ],
        out_specs=[
            pl.BlockSpec((BM, 2048), lambda i: (i, 0)),
            pl.BlockSpec((BM, 2), lambda i: (i, 0)),
            pl.BlockSpec((BM, 2), lambda i: (i, 0)),
            pl.BlockSpec((1, 4), lambda i: (0, 0)),
        ],
        out_shape=[
            jax.ShapeDtypeStruct((NP, 2048), _f32),
            jax.ShapeDtypeStruct((NP, 2), _f32),
            jax.ShapeDtypeStruct((NP, 2), _f32),
            jax.ShapeDtypeStruct((1, 4), _f32),
        ],
    )(x_p, w1, ats, atd)


def _tc2_body(a_ref, h1_ref, b1_ref, w2_ref, ats_ref, atd_ref,
              h2_ref, as_ref, ad_ref, cs_ref):
    i = pl.program_id(0)
    outs = []
    for hd in range(2):
        ah = a_ref[hd]
        num = jnp.dot(ah, h1_ref[:, hd * H1DIM:(hd + 1) * H1DIM],
                      preferred_element_type=_f32)
        den = jnp.sum(ah, axis=1, keepdims=True) + 1e-16
        outs.append(num / den)
    x2 = jnp.concatenate(outs, axis=1) + b1_ref[...]
    x2 = jnp.maximum(x2, 0.0)
    h2 = jnp.dot(x2, w2_ref[...], preferred_element_type=_f32)
    h2_ref[...] = h2
    a_s = jnp.sum(h2 * ats_ref[...], axis=1, keepdims=True)
    a_d = jnp.sum(h2 * atd_ref[...], axis=1, keepdims=True)
    as_ref[...] = a_s
    ad_ref[...] = a_d
    m = jnp.concatenate([jnp.max(a_s, axis=0, keepdims=True),
                         jnp.max(a_d, axis=0, keepdims=True)], axis=1)

    @pl.when(i == 0)
    def _():
        cs_ref[...] = m

    @pl.when(i > 0)
    def _():
        cs_ref[...] = jnp.maximum(cs_ref[...], m)


def _tc2(a1, h1, b1, w2, ats2, atd2):
    return pl.pallas_call(
        _tc2_body,
        grid=(NBLK,),
        in_specs=[
            pl.BlockSpec((2, BM, NP), lambda i: (0, i, 0)),
            pl.BlockSpec((NP, 2048), lambda i: (0, 0)),
            pl.BlockSpec((1, 2048), lambda i: (0, 0)),
            pl.BlockSpec((2048, 64), lambda i: (0, 0)),
            pl.BlockSpec((1, 64), lambda i: (0, 0)),
            pl.BlockSpec((1, 64), lambda i: (0, 0)),
        ],
        out_specs=[
            pl.BlockSpec((BM, 64), lambda i: (i, 0)),
            pl.BlockSpec((BM, 1), lambda i: (i, 0)),
            pl.BlockSpec((BM, 1), lambda i: (i, 0)),
            pl.BlockSpec((1, 2), lambda i: (0, 0)),
        ],
        out_shape=[
            jax.ShapeDtypeStruct((NP, 64), _f32),
            jax.ShapeDtypeStruct((NP, 1), _f32),
            jax.ShapeDtypeStruct((NP, 1), _f32),
            jax.ShapeDtypeStruct((1, 2), _f32),
        ],
    )(a1, h1, b1, w2, ats2, atd2)


def _tc3a_body(a_ref, h2_ref, b2_ref, g_ref):
    ah = a_ref[...]
    num = jnp.dot(ah, h2_ref[...], preferred_element_type=_f32)
    den = jnp.sum(ah, axis=1, keepdims=True) + 1e-16
    g_ref[...] = jnp.maximum(num / den + b2_ref[...], 0.0)


def _tc3a(a2, h2, b2):
    return pl.pallas_call(
        _tc3a_body,
        grid=(NBLK,),
        in_specs=[
            pl.BlockSpec((BM, NP), lambda i: (i, 0)),
            pl.BlockSpec((NP, 64), lambda i: (0, 0)),
            pl.BlockSpec((1, 64), lambda i: (0, 0)),
        ],
        out_specs=pl.BlockSpec((BM, 64), lambda i: (i, 0)),
        out_shape=jax.ShapeDtypeStruct((NP, 64), _f32),
    )(a2, h2, b2)


def _tc3b_body(g_ref, w1_ref, b1_ref, w2_ref, b2_ref, w3_ref, b3_ref,
               w4_ref, b4_ref, o_ref):
    h = jnp.dot(g_ref[...], w1_ref[...], preferred_element_type=_f32)
    h = jnp.maximum(h + b1_ref[...], 0.0)
    h = jnp.dot(h, w2_ref[...], preferred_element_type=_f32)
    h = jnp.maximum(h + b2_ref[...], 0.0)
    h = jnp.dot(h, w3_ref[...], preferred_element_type=_f32)
    h = jnp.maximum(h + b3_ref[...], 0.0)
    h = jnp.dot(h, w4_ref[...], preferred_element_type=_f32)
    o_ref[...] = h + b4_ref[...]


def _tc3b(g5, wf1, bf1, wf2, bf2, wf3, bf3, wf4, bf4):
    return pl.pallas_call(
        _tc3b_body,
        out_shape=jax.ShapeDtypeStruct((5, 10), _f32),
    )(g5, wf1, bf1, wf2, bf2, wf3, bf3, wf4, bf4)


def _lane(v, l, lanes):
    return jnp.max(jnp.where(lanes == l, v, jnp.int32(-2147483647)))


def _dgather(v, idx):
    return lax.gather(
        v, idx[:, None],
        lax.GatherDimensionNumbers(offset_dims=(), collapsed_slice_dims=(0,),
                                   start_index_map=(0,)),
        (1,), mode=lax.GatherScatterMode.PROMISE_IN_BOUNDS)


def _make_sc_build(heads, passes):
    tiles_per_head = NTILES // heads
    rows_per_tile = NP // tiles_per_head
    assert rows_per_tile == passes * R
    mesh = plsc.VectorSubcoreMesh(core_axis_name="c", subcore_axis_name="s")
    nc = 2

    @functools.partial(
        pl.kernel,
        out_type=jax.ShapeDtypeStruct((heads, NP * NP), _f32),
        mesh=mesh,
        scratch_types=[
            pltpu.VMEM((CH,), _i32),
            pltpu.VMEM((CH,), _i32),
            pltpu.VMEM((NP,), _f32),
            pltpu.VMEM((NP,), _f32),
            pltpu.VMEM((16,), _f32),
            pltpu.VMEM((16,), _i32),
            pltpu.VMEM((R * NP,), _f32),
        ],
    )
    def build(src_hbm, dst_hbm, as_hbm, ad_hbm, c_hbm, bnd_hbm, out_hbm,
              sbuf, dbuf, as_v, ad_v, c_v, b_v, acc):
        wid = lax.axis_index("s") * nc + lax.axis_index("c")
        head = wid // tiles_per_head
        tih = wid % tiles_per_head
        pltpu.sync_copy(as_hbm.at[head], as_v)
        pltpu.sync_copy(ad_hbm.at[head], ad_v)
        pltpu.sync_copy(c_hbm.at[head], c_v)
        pltpu.sync_copy(bnd_hbm.at[wid], b_v)
        lanes = lax.iota(_i32, 16)
        bvec = b_v[...]
        cvec = c_v[...]
        zero16 = jnp.zeros((16,), _f32)

        for p in range(passes):
            lo = tih * rows_per_tile + p * R
            c0 = _lane(bvec, 2 * p, lanes)
            c1 = _lane(bvec, 2 * p + 1, lanes)

            def zbody(k, _):
                base = k * 128
                for u in range(8):
                    acc[pl.ds(base + u * 16, 16)] = zero16
                return 0

            lax.fori_loop(0, (R * NP) // 128, zbody, 0)

            def chunk(ci, _):
                pltpu.sync_copy(src_hbm.at[pl.ds(ci * CH, CH)], sbuf)
                pltpu.sync_copy(dst_hbm.at[pl.ds(ci * CH, CH)], dbuf)

                def grp(j, _):
                    s16 = sbuf[pl.ds(j * 16, 16)]
                    d16 = dbuf[pl.ds(j * 16, 16)]
                    a_s = plsc.load_gather(as_v, [s16])
                    a_d = plsc.load_gather(ad_v, [d16])
                    al = a_s + a_d
                    al = jnp.where(al > 0, al, 0.2 * al)
                    e = jnp.exp(al - cvec)
                    key = d16 * NP + s16
                    nxt = _dgather(key, jnp.minimum(lanes + 1, 15))
                    prv = _dgather(key, jnp.maximum(lanes - 1, 0))
                    is_last = (key != nxt) | (lanes == 15)
                    is_first = (key != prv) | (lanes == 0)
                    cs = plsc.cumsum(e)
                    pstart = plsc.cummax(jnp.where(is_first, lanes, 0))
                    csprev = _dgather(cs, jnp.maximum(pstart - 1, 0))
                    val = cs - jnp.where(pstart > 0, csprev, 0.0)
                    m = is_last & (d16 >= lo) & (d16 < lo + R)
                    off = (d16 - lo) * NP + s16
                    off = jnp.where(m, off, 0)
                    plsc.addupdate_scatter(acc, [off], val, mask=m)
                    return 0

                lax.fori_loop(0, CH // 16, grp, 0)
                return 0

            lax.fori_loop(c0, c1, chunk, 0)
            pltpu.sync_copy(acc, out_hbm.at[head, pl.ds(lo * NP, R * NP)])

    return build


_sc_build1 = _make_sc_build(2, 4)
_sc_build2 = _make_sc_build(1, 2)


def _bounds(dsts, heads, passes):
    tiles_per_head = NTILES // heads
    rows_per_tile = NP // tiles_per_head
    los = []
    for wid in range(NTILES):
        tih = wid % tiles_per_head
        for p in range(passes):
            los.append(tih * rows_per_tile + p * R)
    los = jnp.asarray(np.array(los, dtype=np.int32))
    s0 = jnp.searchsorted(dsts, los, side="left").astype(_i32)
    s1 = jnp.searchsorted(dsts, los + R, side="left").astype(_i32)
    c0 = s0 // CH
    c1 = (s1 + CH - 1) // CH
    b = jnp.zeros((NTILES, 16), _i32)
    b = b.at[:, 0:2 * passes:2].set(c0.reshape(NTILES, passes))
    b = b.at[:, 1:2 * passes:2].set(c1.reshape(NTILES, passes))
    return b


def _leaky(x):
    return jnp.where(x > 0, x, 0.2 * x)


def kernel(x, edge_index, W1, att_src1, att_dst1, b1, W2, att_src2, att_dst2,
           b2, Wf1, bf1, Wf2, bf2, Wf3, bf3, Wf4, bf4):
    loop = jnp.arange(N, dtype=edge_index.dtype)
    src = jnp.concatenate([edge_index[0], loop]).astype(_i32)
    dst = jnp.concatenate([edge_index[1], loop]).astype(_i32)
    perm = jnp.argsort(dst * NP + src)
    pad = EP - E1
    srcs = jnp.concatenate([src[perm], jnp.full((pad,), SENT, _i32)])
    dsts = jnp.concatenate([dst[perm], jnp.full((pad,), SENT, _i32)])

    x_p = jnp.pad(x, ((0, NP - N), (0, 0)))
    h1, as1, ad1, cs1 = _tc1(x_p, W1, att_src1, att_dst1)

    c1v = _leaky(cs1[0, :2] + cs1[0, 2:])
    a1 = _sc_build1(srcs, dsts, as1.T, ad1.T,
                    jnp.broadcast_to(c1v[:, None], (2, 16)),
                    _bounds(dsts, 2, 4))

    h2, as2, ad2, cs2 = _tc2(a1.reshape(2, NP, NP), h1, b1.reshape(1, -1),
                             W2, att_src2, att_dst2)

    c2v = _leaky(cs2[0, :1] + cs2[0, 1:])
    a2 = _sc_build2(srcs, dsts, as2.T, ad2.T,
                    jnp.broadcast_to(c2v[:, None], (1, 16)),
                    _bounds(dsts, 1, 2))

    g = _tc3a(a2.reshape(NP, NP), h2, b2.reshape(1, -1))
    g5 = g[:N].reshape(5, 420 * 64)
    return _tc3b(g5, Wf1, bf1.reshape(1, -1), Wf2, bf2.reshape(1, -1),
                 Wf3, bf3.reshape(1, -1), Wf4, bf4.reshape(1, -1))


# speedup vs baseline: 8.7218x; 8.7218x over previous
import functools

import jax
import jax.numpy as jnp
import numpy as np
from jax import lax
from jax.experimental import pallas as pl
from jax.experimental.pallas import tpu as pltpu
from jax.experimental.pallas import tpu_sc as plsc

N = 2100
NP = 2176
E0 = 33600
E1 = E0 + N
CH = 512
EP = ((E1 + CH - 1) // CH) * CH
SENT = 2160
BM = 128
NBLK = NP // BM
H1DIM = 1024
NTILES = 32
R = 34

_f32 = jnp.float32
_i32 = jnp.int32


def _tc1_body(x_ref, w_ref, ats_ref, atd_ref, h_ref, as_ref, ad_ref, cs_ref):
    i = pl.program_id(0)
    h = jnp.dot(x_ref[...], w_ref[...], preferred_element_type=_f32)
    h_ref[...] = h
    cols = []
    for hd in range(2):
        hh = h[:, hd * H1DIM:(hd + 1) * H1DIM]
        a_s = jnp.sum(hh * ats_ref[hd:hd + 1, :], axis=1, keepdims=True)
        a_d = jnp.sum(hh * atd_ref[hd:hd + 1, :], axis=1, keepdims=True)
        cols.append((a_s, a_d))
    a_s = jnp.concatenate([cols[0][0], cols[1][0]], axis=1)
    a_d = jnp.concatenate([cols[0][1], cols[1][1]], axis=1)
    as_ref[...] = a_s
    ad_ref[...] = a_d
    m = jnp.concatenate([jnp.max(a_s, axis=0, keepdims=True),
                         jnp.max(a_d, axis=0, keepdims=True)], axis=1)

    @pl.when(i == 0)
    def _():
        cs_ref[...] = m

    @pl.when(i > 0)
    def _():
        cs_ref[...] = jnp.maximum(cs_ref[...], m)


def _tc1(x_p, w1, ats, atd):
    return pl.pallas_call(
        _tc1_body,
        grid=(NBLK,),
        in_specs=[
            pl.BlockSpec((BM, 2048), lambda i: (i, 0)),
            pl.BlockSpec((2048, 2048), lambda i: (0, 0)),
            pl.BlockSpec((2, H1DIM), lambda i: (0, 0)),
            pl.BlockSpec((2, H1DIM), lambda i: (0, 0)),
        ],
        out_specs=[
            pl.BlockSpec((BM, 2048), lambda i: (i, 0)),
            pl.BlockSpec((BM, 2), lambda i: (i, 0)),
            pl.BlockSpec((BM, 2), lambda i: (i, 0)),
            pl.BlockSpec((1, 4), lambda i: (0, 0)),
        ],
        out_shape=[
            jax.ShapeDtypeStruct((NP, 2048), _f32),
            jax.ShapeDtypeStruct((NP, 2), _f32),
            jax.ShapeDtypeStruct((NP, 2), _f32),
            jax.ShapeDtypeStruct((1, 4), _f32),
        ],
    )(x_p, w1, ats, atd)


def _tc2_body(a_ref, h1_ref, b1_ref, w2_ref, ats_ref, atd_ref,
              h2_ref, as_ref, ad_ref, cs_ref):
    i = pl.program_id(0)
    outs = []
    for hd in range(2):
        ah = a_ref[hd]
        num = jnp.dot(ah, h1_ref[:, hd * H1DIM:(hd + 1) * H1DIM],
                      preferred_element_type=_f32)
        den = jnp.sum(ah, axis=1, keepdims=True) + 1e-16
        outs.append(num / den)
    x2 = jnp.concatenate(outs, axis=1) + b1_ref[...]
    x2 = jnp.maximum(x2, 0.0)
    h2 = jnp.dot(x2, w2_ref[...], preferred_element_type=_f32)
    h2_ref[...] = h2
    a_s = jnp.sum(h2 * ats_ref[...], axis=1, keepdims=True)
    a_d = jnp.sum(h2 * atd_ref[...], axis=1, keepdims=True)
    as_ref[...] = a_s
    ad_ref[...] = a_d
    m = jnp.concatenate([jnp.max(a_s, axis=0, keepdims=True),
                         jnp.max(a_d, axis=0, keepdims=True)], axis=1)

    @pl.when(i == 0)
    def _():
        cs_ref[...] = m

    @pl.when(i > 0)
    def _():
        cs_ref[...] = jnp.maximum(cs_ref[...], m)


def _tc2(a1, h1, b1, w2, ats2, atd2):
    return pl.pallas_call(
        _tc2_body,
        grid=(NBLK,),
        in_specs=[
            pl.BlockSpec((2, BM, NP), lambda i: (0, i, 0)),
            pl.BlockSpec((NP, 2048), lambda i: (0, 0)),
            pl.BlockSpec((1, 2048), lambda i: (0, 0)),
            pl.BlockSpec((2048, 64), lambda i: (0, 0)),
            pl.BlockSpec((1, 64), lambda i: (0, 0)),
            pl.BlockSpec((1, 64), lambda i: (0, 0)),
        ],
        out_specs=[
            pl.BlockSpec((BM, 64), lambda i: (i, 0)),
            pl.BlockSpec((BM, 1), lambda i: (i, 0)),
            pl.BlockSpec((BM, 1), lambda i: (i, 0)),
            pl.BlockSpec((1, 2), lambda i: (0, 0)),
        ],
        out_shape=[
            jax.ShapeDtypeStruct((NP, 64), _f32),
            jax.ShapeDtypeStruct((NP, 1), _f32),
            jax.ShapeDtypeStruct((NP, 1), _f32),
            jax.ShapeDtypeStruct((1, 2), _f32),
        ],
    )(a1, h1, b1, w2, ats2, atd2)


def _tc3a_body(a_ref, h2_ref, b2_ref, g_ref):
    ah = a_ref[...]
    num = jnp.dot(ah, h2_ref[...], preferred_element_type=_f32)
    den = jnp.sum(ah, axis=1, keepdims=True) + 1e-16
    g_ref[...] = jnp.maximum(num / den + b2_ref[...], 0.0)


def _tc3a(a2, h2, b2):
    return pl.pallas_call(
        _tc3a_body,
        grid=(NBLK,),
        in_specs=[
            pl.BlockSpec((BM, NP), lambda i: (i, 0)),
            pl.BlockSpec((NP, 64), lambda i: (0, 0)),
            pl.BlockSpec((1, 64), lambda i: (0, 0)),
        ],
        out_specs=pl.BlockSpec((BM, 64), lambda i: (i, 0)),
        out_shape=jax.ShapeDtypeStruct((NP, 64), _f32),
    )(a2, h2, b2)


def _tc3b_body(g_ref, w1_ref, b1_ref, w2_ref, b2_ref, w3_ref, b3_ref,
               w4_ref, b4_ref, o_ref):
    h = jnp.dot(g_ref[...], w1_ref[...], preferred_element_type=_f32)
    h = jnp.maximum(h + b1_ref[...], 0.0)
    h = jnp.dot(h, w2_ref[...], preferred_element_type=_f32)
    h = jnp.maximum(h + b2_ref[...], 0.0)
    h = jnp.dot(h, w3_ref[...], preferred_element_type=_f32)
    h = jnp.maximum(h + b3_ref[...], 0.0)
    h = jnp.dot(h, w4_ref[...], preferred_element_type=_f32)
    o_ref[...] = h + b4_ref[...]


def _tc3b(g5, wf1, bf1, wf2, bf2, wf3, bf3, wf4, bf4):
    return pl.pallas_call(
        _tc3b_body,
        out_shape=jax.ShapeDtypeStruct((5, 10), _f32),
    )(g5, wf1, bf1, wf2, bf2, wf3, bf3, wf4, bf4)


def _make_sc_build(heads, passes):
    tiles_per_head = NTILES // heads
    rows_per_tile = NP // tiles_per_head
    assert rows_per_tile == passes * R
    mesh = plsc.VectorSubcoreMesh(core_axis_name="c", subcore_axis_name="s",
                                  num_cores=2, num_subcores=16)
    nc = 2

    @functools.partial(
        pl.kernel,
        out_type=jax.ShapeDtypeStruct((heads, NP * NP), _f32),
        mesh=mesh,
        compiler_params=pltpu.CompilerParams(needs_layout_passes=False),
        scratch_types=[
            pltpu.VMEM((CH + 16,), _i32),
            pltpu.VMEM((CH + 16,), _i32),
            pltpu.VMEM((NP,), _f32),
            pltpu.VMEM((NP,), _f32),
            pltpu.VMEM((16,), _f32),
            pltpu.VMEM((16,), _i32),
            pltpu.VMEM((R * NP,), _f32),
        ],
    )
    def build(src_hbm, dst_hbm, as_hbm, ad_hbm, c_hbm, bnd_hbm, out_hbm,
              sbuf, dbuf, as_v, ad_v, c_v, b_v, acc):
        wid = lax.axis_index("s") * nc + lax.axis_index("c")
        head = wid // tiles_per_head
        tih = wid % tiles_per_head
        pltpu.sync_copy(as_hbm.at[head], as_v)
        pltpu.sync_copy(ad_hbm.at[head], ad_v)
        pltpu.sync_copy(c_hbm.at[head], c_v)
        pltpu.sync_copy(bnd_hbm.at[wid], b_v)
        lanes = lax.iota(_i32, 16)
        cvec = c_v[...]
        zero16 = jnp.zeros((16,), _f32)
        sbuf[pl.ds(CH, 16)] = lanes
        dbuf[pl.ds(CH, 16)] = lanes

        bvec = b_v[...]
        for p in range(passes):
            lo = tih * rows_per_tile + p * R
            c0 = bvec[2 * p]
            c1 = bvec[2 * p + 1]

            def zbody(k, _):
                base = k * 128
                for u in range(8):
                    acc[pl.ds(base + u * 16, 16)] = zero16
                return 0

            lax.fori_loop(0, (R * NP) // 128, zbody, 0)

            def chunk(ci, _):
                pltpu.sync_copy(src_hbm.at[pl.ds(ci * CH, CH)],
                                sbuf.at[pl.ds(0, CH)])
                pltpu.sync_copy(dst_hbm.at[pl.ds(ci * CH, CH)],
                                dbuf.at[pl.ds(0, CH)])

                def grp(j, _):
                    s16 = sbuf[pl.ds(j * 16, 16)]
                    d16 = dbuf[pl.ds(j * 16, 16)]
                    s17 = sbuf[pl.ds(j * 16 + 1, 16)]
                    d17 = dbuf[pl.ds(j * 16 + 1, 16)]
                    a_s = plsc.load_gather(as_v, [s16])
                    a_d = plsc.load_gather(ad_v, [d16])
                    al = a_s + a_d
                    al = jnp.where(al > 0, al, 0.2 * al)
                    e = jnp.exp(al - cvec)
                    m = (d16 >= lo) & (d16 < lo + R)
                    off = jnp.where(m, (d16 - lo) * NP + s16, 0)
                    adj = (d16 == d17) & (s16 == s17) & (lanes < 15) & m
                    ndup = plsc.all_reduce_population_count(adj)[0]

                    def fast():
                        plsc.addupdate_scatter(acc, [off], e, mask=m)

                    def slow():
                        def one(t, _):
                            plsc.addupdate_scatter(acc, [off], e,
                                                   mask=m & (lanes == t))
                            return 0
                        lax.fori_loop(0, 16, one, 0)

                    lax.cond(ndup == 0, fast, slow)
                    return 0

                lax.fori_loop(0, CH // 16, grp, 0)
                return 0

            lax.fori_loop(c0, c1, chunk, 0)
            pltpu.sync_copy(acc, out_hbm.at[head, pl.ds(lo * NP, R * NP)])

    return build


_sc_build1 = _make_sc_build(2, 4)
_sc_build2 = _make_sc_build(1, 2)


def _bounds(dsts, heads, passes):
    tiles_per_head = NTILES // heads
    rows_per_tile = NP // tiles_per_head
    los = []
    for wid in range(NTILES):
        tih = wid % tiles_per_head
        for p in range(passes):
            los.append(tih * rows_per_tile + p * R)
    los = jnp.asarray(np.array(los, dtype=np.int32))
    s0 = jnp.searchsorted(dsts, los, side="left").astype(_i32)
    s1 = jnp.searchsorted(dsts, los + R, side="left").astype(_i32)
    c0 = s0 // CH
    c1 = (s1 + CH - 1) // CH
    b = jnp.zeros((NTILES, 16), _i32)
    b = b.at[:, 0:2 * passes:2].set(c0.reshape(NTILES, passes))
    b = b.at[:, 1:2 * passes:2].set(c1.reshape(NTILES, passes))
    return b


def _leaky(x):
    return jnp.where(x > 0, x, 0.2 * x)


def kernel(x, edge_index, W1, att_src1, att_dst1, b1, W2, att_src2, att_dst2,
           b2, Wf1, bf1, Wf2, bf2, Wf3, bf3, Wf4, bf4):
    loop = jnp.arange(N, dtype=edge_index.dtype)
    src = jnp.concatenate([edge_index[0], loop]).astype(_i32)
    dst = jnp.concatenate([edge_index[1], loop]).astype(_i32)
    perm = jnp.argsort(dst * NP + src)
    pad = EP - E1
    srcs = jnp.concatenate([src[perm], jnp.full((pad,), SENT, _i32)])
    dsts = jnp.concatenate([dst[perm], jnp.full((pad,), SENT, _i32)])

    x_p = jnp.pad(x, ((0, NP - N), (0, 0)))
    h1, as1, ad1, cs1 = _tc1(x_p, W1, att_src1, att_dst1)

    c1v = _leaky(cs1[0, :2] + cs1[0, 2:])
    a1 = _sc_build1(srcs, dsts, as1.T, ad1.T,
                    jnp.broadcast_to(c1v[:, None], (2, 16)),
                    _bounds(dsts, 2, 4))

    h2, as2, ad2, cs2 = _tc2(a1.reshape(2, NP, NP), h1, b1.reshape(1, -1),
                             W2, att_src2, att_dst2)

    c2v = _leaky(cs2[0, :1] + cs2[0, 1:])
    a2 = _sc_build2(srcs, dsts, as2.T, ad2.T,
                    jnp.broadcast_to(c2v[:, None], (1, 16)),
                    _bounds(dsts, 1, 2))

    g = _tc3a(a2.reshape(NP, NP), h2, b2.reshape(1, -1))
    g5 = g[:N].reshape(5, 420 * 64)
    return _tc3b(g5, Wf1, bf1.reshape(1, -1), Wf2, bf2.reshape(1, -1),
                 Wf3, bf3.reshape(1, -1), Wf4, bf4.reshape(1, -1))
